# baseline (speedup 1.0000x reference)
import functools

import jax
import jax.numpy as jnp
import numpy as np
from jax import lax
from jax.experimental import pallas as pl
from jax.experimental.pallas import tpu as pltpu
from jax.experimental.pallas import tpu_sc as plsc

VOCAB = 100000
HIDDEN = 1024
BATCH = 4
SEQ = 2048

NC = 2
NS = 16
NW = NC * NS
TOTAL = BATCH * SEQ
NPIECE = 2
PIECE = TOTAL // NPIECE
ROWS_PER_W = PIECE // NW
CHUNK = 32
NCHUNK = ROWS_PER_W // CHUNK
NBUF = 3
LOOKAHEAD = 2

BLK = 256
PIECE_BLKS = PIECE // BLK
PE_BLKS = SEQ // BLK


def _pos_encoding() -> np.ndarray:
    pos = np.arange(SEQ)[:, None].astype(np.float64)
    i = np.arange(HIDDEN // 2)[None, :].astype(np.float64)
    angle = pos / np.power(10000.0, 2.0 * i / HIDDEN)
    pe = np.zeros((SEQ, HIDDEN), dtype=np.float64)
    pe[:, 0::2] = np.sin(angle)
    pe[:, 1::2] = np.cos(angle)
    return pe.astype(np.float32)


_PE = _pos_encoding()


def _gather_body(ids_hbm, table_hbm, out_hbm, idx_v, buf0, buf1, buf2,
                 g0, g1, g2, o0, o1, o2):
    c = lax.axis_index("c")
    s = lax.axis_index("s")
    wid = s * NC + c
    base = wid * ROWS_PER_W

    bufs = (buf0, buf1, buf2)
    gsems = (g0, g1, g2)
    osems = (o0, o1, o2)

    pltpu.sync_copy(ids_hbm.at[wid], idx_v)

    gather_d = [None] * NBUF
    out_d = [None] * NBUF

    for t in range(NCHUNK + LOOKAHEAD):
        if t < NCHUNK:
            k = t % NBUF
            if out_d[k] is not None:
                out_d[k].wait()
            gather_d[k] = pltpu.async_copy(
                table_hbm.at[idx_v.at[t]], bufs[k], gsems[k])
        if t >= LOOKAHEAD:
            ch = t - LOOKAHEAD
            k = ch % NBUF
            gather_d[k].wait()
            out_d[k] = pltpu.async_copy(
                bufs[k], out_hbm.at[pl.ds(base + ch * CHUNK, CHUNK)],
                osems[k])

    for k in range(NBUF):
        if out_d[k] is not None:
            out_d[k].wait()


def _sc_gather(ids3, table):
    mesh = plsc.VectorSubcoreMesh(core_axis_name="c", subcore_axis_name="s")
    f = pl.kernel(
        _gather_body,
        out_type=jax.ShapeDtypeStruct((PIECE, HIDDEN), jnp.float32),
        mesh=mesh,
        scratch_types=(
            [pltpu.VMEM((NCHUNK, CHUNK), jnp.int32)]
            + [pltpu.VMEM((CHUNK, HIDDEN), jnp.float32)] * NBUF
            + [pltpu.SemaphoreType.DMA] * (2 * NBUF)
        ),
    )
    return f(ids3, table)


def _add0_body(g_ref, pe_ref, out_ref):
    out_ref[...] = g_ref[...] + pe_ref[...]


def _add1_body(big_ref, g_ref, pe_ref, out_ref):
    del big_ref
    out_ref[...] = g_ref[...] + pe_ref[...]


def _tc_add0(g0, pe):
    return pl.pallas_call(
        _add0_body,
        grid=(PIECE_BLKS,),
        in_specs=[
            pl.BlockSpec((BLK, HIDDEN), lambda i: (i, 0)),
            pl.BlockSpec((BLK, HIDDEN), lambda i: (lax.rem(i, PE_BLKS), 0)),
        ],
        out_specs=pl.BlockSpec((BLK, HIDDEN), lambda i: (i, 0)),
        out_shape=jax.ShapeDtypeStruct((TOTAL, HIDDEN), jnp.float32),
    )(g0, pe)


def _tc_add1(big, g1, pe):
    return pl.pallas_call(
        _add1_body,
        grid=(PIECE_BLKS,),
        in_specs=[
            pl.BlockSpec(memory_space=pltpu.MemorySpace.HBM),
            pl.BlockSpec((BLK, HIDDEN), lambda i: (i, 0)),
            pl.BlockSpec((BLK, HIDDEN), lambda i: (lax.rem(i, PE_BLKS), 0)),
        ],
        out_specs=pl.BlockSpec((BLK, HIDDEN), lambda i: (PIECE_BLKS + i, 0)),
        out_shape=jax.ShapeDtypeStruct((TOTAL, HIDDEN), jnp.float32),
        input_output_aliases={0: 0},
    )(big, g1, pe)


@jax.jit
def _embed(ids3_0, ids3_1, pe, table):
    g0 = _sc_gather(ids3_0, table)
    g1 = _sc_gather(ids3_1, table)
    big = _tc_add0(g0, pe)
    big = _tc_add1(big, g1, pe)
    return big


def kernel(input_ids, token_embed_weight):
    ids = input_ids.astype(jnp.int32).reshape(NPIECE, NW, NCHUNK, CHUNK)
    pe = jnp.asarray(_PE)
    out = _embed(ids[0], ids[1], pe, token_embed_weight)
    return out.reshape(BATCH, SEQ, HIDDEN)
